# baseline (speedup 1.0000x reference)
import functools

import jax
import jax.numpy as jnp
from jax import lax
from jax.experimental import pallas as pl
from jax.experimental.pallas import tpu as pltpu
from jax.experimental.pallas import tpu_sc as plsc

N = 10000
E = 320000
EP = 327680
D = 128
NC, NS = 2, 16
NW = NC * NS
EPW = EP // NW
C = 32
UNROLL = 4
NCHUNK = EPW // C
NPAD = 10240
BN = 2000
SUPER = 16
SCSZ = SUPER * C


def _proj_body(ske, typ, loc, h, w1s, w1t, w1l, b1, w2, b2,
               mqd, mqs, mpd, mps, td, ts):
    e1 = ske[...] @ w1s[...] + typ[...] @ w1t[...] + loc[...] @ w1l[...]
    e1 = jnp.maximum(e1 + b1[...], 0.0)
    emb = jnp.maximum(e1 @ w2[...] + b2[...], 0.0)
    hb = h[...]
    td[...] = jnp.concatenate([hb @ mqd[...], emb @ mpd[...]], axis=1)
    ts[...] = jnp.concatenate([hb @ mqs[...], emb @ mps[...]], axis=1)


_proj = pl.pallas_call(
    _proj_body,
    grid=(N // BN,),
    in_specs=[
        pl.BlockSpec((BN, D), lambda i: (i, 0)),
        pl.BlockSpec((BN, 16), lambda i: (i, 0)),
        pl.BlockSpec((BN, D), lambda i: (i, 0)),
        pl.BlockSpec((BN, D), lambda i: (i, 0)),
        pl.BlockSpec((D, D), lambda i: (0, 0)),
        pl.BlockSpec((16, D), lambda i: (0, 0)),
        pl.BlockSpec((D, D), lambda i: (0, 0)),
        pl.BlockSpec((1, D), lambda i: (0, 0)),
        pl.BlockSpec((D, D), lambda i: (0, 0)),
        pl.BlockSpec((1, D), lambda i: (0, 0)),
        pl.BlockSpec((D, D), lambda i: (0, 0)),
        pl.BlockSpec((D, D), lambda i: (0, 0)),
        pl.BlockSpec((D, D), lambda i: (0, 0)),
        pl.BlockSpec((D, D), lambda i: (0, 0)),
    ],
    out_specs=[
        pl.BlockSpec((BN, 2 * D), lambda i: (i, 0)),
        pl.BlockSpec((BN, 2 * D), lambda i: (i, 0)),
    ],
    out_shape=[
        jax.ShapeDtypeStruct((N, 2 * D), jnp.float32),
        jax.ShapeDtypeStruct((N, 2 * D), jnp.float32),
    ],
)


_mesh = plsc.VectorSubcoreMesh(core_axis_name="c", subcore_axis_name="s",
                               num_cores=NC, num_subcores=NS)


@functools.partial(
    pl.kernel,
    out_type=[
        jax.ShapeDtypeStruct((NC, NPAD, D), jnp.float32),
        jax.ShapeDtypeStruct((NC, NS, NPAD), jnp.float32),
    ],
    mesh=_mesh,
    scratch_types=[
        pltpu.VMEM((C,), jnp.int32),
        pltpu.VMEM((C,), jnp.int32),
        pltpu.VMEM((C,), jnp.int32),
        pltpu.VMEM((C,), jnp.int32),
        pltpu.VMEM((SCSZ,), jnp.int32),
        pltpu.VMEM((SCSZ,), jnp.int32),
        pltpu.VMEM((C, 2 * D), jnp.float32),
        pltpu.VMEM((C, 2 * D), jnp.float32),
        pltpu.VMEM((C, D), jnp.float32),
        pltpu.VMEM((C,), jnp.float32),
        pltpu.VMEM((NPAD,), jnp.float32),
        pltpu.VMEM((D,), jnp.float32),
        pltpu.VMEM_SHARED((NPAD, D), jnp.float32),
        pltpu.SemaphoreType.DMA,
        pltpu.SemaphoreType.DMA,
    ],
    compiler_params=pltpu.CompilerParams(needs_layout_passes=False),
)
def _edge_pass(td_hbm, ts_hbm, dst_hbm, src_hbm, wa2_hbm, zero_hbm,
               out_msg, out_den,
               idx_d0, idx_d1, idx_s0, idx_s1, sdx, ssx,
               ra, rb, obuf, exs_v, den_v, wa2_v, msg_tab, sem_g, sem_sm):
    cid = lax.axis_index("c")
    sid = lax.axis_index("s")
    wid = cid * NS + sid

    pltpu.sync_copy(zero_hbm.at[pl.ds(sid * (NPAD // NS), NPAD // NS)],
                    msg_tab.at[pl.ds(sid * (NPAD // NS), NPAD // NS)])
    pltpu.sync_copy(wa2_hbm, wa2_v)

    zero16 = jnp.zeros((16,), jnp.float32)

    def zero_den(i, c):
        den_v[pl.ds(16 * i, 16)] = zero16
        return c
    lax.fori_loop(0, NPAD // 16, zero_den, 0)

    def zero_obuf(i, c):
        for j in range(8):
            obuf[i, pl.ds(16 * j, 16)] = zero16
        return c
    lax.fori_loop(0, C, zero_obuf, 0)

    plsc.subcore_barrier()

    wa2v = [wa2_v[pl.ds(16 * j, 16)] for j in range(8)]
    lane = lax.iota(jnp.int32, 16)
    lane0 = lane == 0
    perms = [(lane + s) & 15 for s in (8, 4, 2, 1)]
    idxb = [(idx_d0, idx_s0), (idx_d1, idx_s1)]

    def do_chunk(j, p):
        idxd, idxs = idxb[p]
        for g in range(C // 16):
            idxd[pl.ds(16 * g, 16)] = sdx[pl.ds(j * C + 16 * g, 16)]
            idxs[pl.ds(16 * g, 16)] = ssx[pl.ds(j * C + 16 * g, 16)]

        pltpu.async_copy(td_hbm.at[idxd], ra, sem_g)
        pltpu.async_copy(ts_hbm.at[idxs], rb, sem_g)
        pltpu.make_async_copy(zero_hbm.at[pl.ds(0, C)], obuf, sem_sm).wait()
        pltpu.make_async_copy(td_hbm.at[pl.ds(0, C)], ra, sem_g).wait()
        pltpu.make_async_copy(td_hbm.at[pl.ds(0, C)], rb, sem_g).wait()

        for g in range(C // 16):
            rowi = lane + 16 * g

            def feat_body(f, accs):
                a0, a1 = accs
                for fj in range(2):
                    j = 2 * f + fj
                    jcast = jnp.full((16,), j, jnp.int32)
                    ta = plsc.load_gather(ra, [rowi, jcast])
                    tb = plsc.load_gather(rb, [rowi, jcast])
                    t = ta + tb
                    t = jnp.where(t > 0.0, t, 0.01 * t)
                    wv = plsc.load_gather(wa2_v, [jcast])
                    if fj == 0:
                        a0 = a0 + t * wv
                    else:
                        a1 = a1 + t * wv
                return a0, a1

            z16 = jnp.zeros((16,), jnp.float32)
            a0, a1 = lax.fori_loop(0, D // 2, feat_body, (z16, z16))
            exs_v[pl.ds(16 * g, 16)] = jnp.exp(a0 + a1)

        def edge_grp(i, c2):
            for u in range(UNROLL):
                e = UNROLL * i + u
                ecast = jnp.full((16,), e, jnp.int32)
                dstv = plsc.load_gather(idxd, [ecast])
                exv = plsc.load_gather(exs_v, [ecast])
                plsc.addupdate_scatter(den_v, [dstv], exv, mask=lane0)
                for j2 in range(8):
                    u2 = (ra[e, pl.ds(D + 16 * j2, 16)]
                          + rb[e, pl.ds(D + 16 * j2, 16)])
                    obuf[e, pl.ds(16 * j2, 16)] = jnp.maximum(u2, 0.0) * exv
            return c2

        lax.fori_loop(0, C // UNROLL, edge_grp, 0)

        pltpu.async_copy(obuf, msg_tab.at[idxd], sem_sm, add=True)

    pltpu.sync_copy(dst_hbm.at[pl.ds(wid * EPW, C)], idx_d1)
    pltpu.async_copy(obuf, msg_tab.at[idx_d1], sem_sm, add=True)

    def super_body(s, carry):
        base = wid * EPW + s * SCSZ
        pltpu.sync_copy(dst_hbm.at[pl.ds(base, SCSZ)], sdx)
        pltpu.sync_copy(src_hbm.at[pl.ds(base, SCSZ)], ssx)

        def pair_body(i, c2):
            do_chunk(2 * i, 0)
            do_chunk(2 * i + 1, 1)
            return c2

        lax.fori_loop(0, SUPER // 2, pair_body, 0)
        return carry

    lax.fori_loop(0, NCHUNK // SUPER, super_body, 0)
    pltpu.make_async_copy(zero_hbm.at[pl.ds(0, C)], obuf, sem_sm).wait()

    plsc.subcore_barrier()
    pltpu.sync_copy(msg_tab.at[pl.ds(sid * (NPAD // NS), NPAD // NS)],
                    out_msg.at[cid, pl.ds(sid * (NPAD // NS), NPAD // NS)])
    pltpu.sync_copy(den_v, out_den.at[cid, sid])


def _dsum_body(den, out):
    out[...] = jnp.sum(den[...], axis=0)


_dsum = pl.pallas_call(
    _dsum_body,
    grid=(1,),
    in_specs=[pl.BlockSpec((NW, NPAD // D, D), lambda i: (0, 0, 0))],
    out_specs=pl.BlockSpec((NPAD // D, D), lambda i: (0, 0)),
    out_shape=jax.ShapeDtypeStruct((NPAD // D, D), jnp.float32),
)


def _update_body(msg, den, h, wih, whh, bih, bhh, wout, bout, out):
    a = msg[0] + msg[1]
    d = den[...]
    agg = a / (d + 1e-9)
    gi = agg @ wih[...] + bih[...]
    gh = h[...] @ whh[...] + bhh[...]
    r = jax.nn.sigmoid(gi[:, :D] + gh[:, :D])
    z = jax.nn.sigmoid(gi[:, D:2 * D] + gh[:, D:2 * D])
    n = jnp.tanh(gi[:, 2 * D:] + r * gh[:, 2 * D:])
    hn = (1.0 - z) * n + z * h[...]
    out[...] = jnp.maximum(hn @ wout[...] + bout[...], 0.0)


_update = pl.pallas_call(
    _update_body,
    grid=(N // BN,),
    in_specs=[
        pl.BlockSpec((NC, BN, D), lambda i: (0, i, 0)),
        pl.BlockSpec((BN, 1), lambda i: (i, 0)),
        pl.BlockSpec((BN, D), lambda i: (i, 0)),
        pl.BlockSpec((D, 3 * D), lambda i: (0, 0)),
        pl.BlockSpec((D, 3 * D), lambda i: (0, 0)),
        pl.BlockSpec((1, 3 * D), lambda i: (0, 0)),
        pl.BlockSpec((1, 3 * D), lambda i: (0, 0)),
        pl.BlockSpec((D, D), lambda i: (0, 0)),
        pl.BlockSpec((1, D), lambda i: (0, 0)),
    ],
    out_specs=pl.BlockSpec((BN, D), lambda i: (i, 0)),
    out_shape=jax.ShapeDtypeStruct((N, D), jnp.float32),
)


def kernel(obj_loc, obj_ske, obj_type, h, edge_index, W_e1, b_e1, W_e2, b_e2,
           Wa1, Wa2, Ww, W_ih, W_hh, b_ih, b_hh, W_out, b_out):
    ei = edge_index.astype(jnp.int32)
    pad = jnp.full((EP - E,), N + 100, jnp.int32)
    src_i = jnp.concatenate([ei[0], pad])
    dst_i = jnp.concatenate([ei[1], pad])

    w1 = W_e1.T
    w1s, w1t, w1l = w1[:D], w1[D:D + 16], w1[D + 16:]
    b1 = b_e1.reshape(1, D)
    b2 = b_e2.reshape(1, D)
    mqd = (Wa1[:, :D] + Wa1[:, D:]).T
    mqs = -Wa1[:, D:].T
    mpd = (Ww[:, :D] + Ww[:, D:]).T
    mps = -Ww[:, D:].T

    td, ts = _proj(obj_ske, obj_type, obj_loc, h, w1s, w1t, w1l,
                   b1, W_e2.T, b2, mqd, mqs, mpd, mps)
    padrows = jnp.zeros((NPAD - N, 2 * D), jnp.float32)
    td = jnp.concatenate([td, padrows])
    ts = jnp.concatenate([ts, padrows])

    wa2 = Wa2.reshape(D)
    zeros = jnp.zeros((NPAD, D), jnp.float32)
    msg, den = _edge_pass(td, ts, dst_i, src_i, wa2, zeros)
    dsum = _dsum(den.reshape(NW, NPAD // D, D))
    den2 = dsum.reshape(NPAD, 1)

    return _update(msg, den2, h, W_ih.T, W_hh.T, b_ih.reshape(1, 3 * D),
                   b_hh.reshape(1, 3 * D), W_out.T, b_out.reshape(1, D))


# speedup vs baseline: 1.6644x; 1.6644x over previous
import functools

import jax
import jax.numpy as jnp
from jax import lax
from jax.experimental import pallas as pl
from jax.experimental.pallas import tpu as pltpu
from jax.experimental.pallas import tpu_sc as plsc

N = 10000
E = 320000
EP = 327680
D = 128
NC, NS = 2, 16
NW = NC * NS
EPW = EP // NW
C = 32
UNROLL = 4
NCHUNK = EPW // C
NPAD = 10240
BN = 2000
SUPER = 16
SCSZ = SUPER * C


def _proj_body(ske, typ, loc, h, w1s, w1t, w1l, b1, w2, b2,
               mqd, mqs, mpd, mps, td, ts):
    e1 = ske[...] @ w1s[...] + typ[...] @ w1t[...] + loc[...] @ w1l[...]
    e1 = jnp.maximum(e1 + b1[...], 0.0)
    emb = jnp.maximum(e1 @ w2[...] + b2[...], 0.0)
    hb = h[...]
    td[...] = jnp.concatenate([hb @ mqd[...], emb @ mpd[...]], axis=1)
    ts[...] = jnp.concatenate([hb @ mqs[...], emb @ mps[...]], axis=1)


_proj = pl.pallas_call(
    _proj_body,
    grid=(N // BN,),
    in_specs=[
        pl.BlockSpec((BN, D), lambda i: (i, 0)),
        pl.BlockSpec((BN, 16), lambda i: (i, 0)),
        pl.BlockSpec((BN, D), lambda i: (i, 0)),
        pl.BlockSpec((BN, D), lambda i: (i, 0)),
        pl.BlockSpec((D, D), lambda i: (0, 0)),
        pl.BlockSpec((16, D), lambda i: (0, 0)),
        pl.BlockSpec((D, D), lambda i: (0, 0)),
        pl.BlockSpec((1, D), lambda i: (0, 0)),
        pl.BlockSpec((D, D), lambda i: (0, 0)),
        pl.BlockSpec((1, D), lambda i: (0, 0)),
        pl.BlockSpec((D, D), lambda i: (0, 0)),
        pl.BlockSpec((D, D), lambda i: (0, 0)),
        pl.BlockSpec((D, D), lambda i: (0, 0)),
        pl.BlockSpec((D, D), lambda i: (0, 0)),
    ],
    out_specs=[
        pl.BlockSpec((BN, 2 * D), lambda i: (i, 0)),
        pl.BlockSpec((BN, 2 * D), lambda i: (i, 0)),
    ],
    out_shape=[
        jax.ShapeDtypeStruct((N, 2 * D), jnp.float32),
        jax.ShapeDtypeStruct((N, 2 * D), jnp.float32),
    ],
)


_mesh = plsc.VectorSubcoreMesh(core_axis_name="c", subcore_axis_name="s",
                               num_cores=NC, num_subcores=NS)


@functools.partial(
    pl.kernel,
    out_type=[
        jax.ShapeDtypeStruct((NC, NPAD, D), jnp.float32),
        jax.ShapeDtypeStruct((NC, NS, NPAD), jnp.float32),
    ],
    mesh=_mesh,
    scratch_types=[
        pltpu.VMEM((C,), jnp.int32),
        pltpu.VMEM((C,), jnp.int32),
        pltpu.VMEM((C,), jnp.int32),
        pltpu.VMEM((C,), jnp.int32),
        pltpu.VMEM((SCSZ,), jnp.int32),
        pltpu.VMEM((SCSZ,), jnp.int32),
        pltpu.VMEM((C, 2 * D), jnp.float32),
        pltpu.VMEM((C, 2 * D), jnp.float32),
        pltpu.VMEM((C, D), jnp.float32),
        pltpu.VMEM((NPAD,), jnp.float32),
        pltpu.VMEM((D,), jnp.float32),
        pltpu.VMEM_SHARED((NPAD, D), jnp.float32),
        pltpu.SemaphoreType.DMA,
        pltpu.SemaphoreType.DMA,
    ],
    compiler_params=pltpu.CompilerParams(needs_layout_passes=False),
)
def _edge_pass(td_hbm, ts_hbm, dst_hbm, src_hbm, wa2_hbm, zero_hbm,
               out_msg, out_den,
               idx_d0, idx_d1, idx_s0, idx_s1, sdx, ssx,
               ra, rb, obuf, den_v, wa2_v, msg_tab, sem_g, sem_sm):
    cid = lax.axis_index("c")
    sid = lax.axis_index("s")
    wid = cid * NS + sid

    pltpu.sync_copy(zero_hbm.at[pl.ds(sid * (NPAD // NS), NPAD // NS)],
                    msg_tab.at[pl.ds(sid * (NPAD // NS), NPAD // NS)])
    pltpu.sync_copy(wa2_hbm, wa2_v)

    zero16 = jnp.zeros((16,), jnp.float32)

    def zero_den(i, c):
        den_v[pl.ds(16 * i, 16)] = zero16
        return c
    lax.fori_loop(0, NPAD // 16, zero_den, 0)

    def zero_obuf(i, c):
        for j in range(8):
            obuf[i, pl.ds(16 * j, 16)] = zero16
        return c
    lax.fori_loop(0, C, zero_obuf, 0)

    plsc.subcore_barrier()

    wa2v = [wa2_v[pl.ds(16 * j, 16)] for j in range(8)]
    lane = lax.iota(jnp.int32, 16)
    lane0 = lane == 0
    perms = [(lane + s) & 15 for s in (8, 4, 2, 1)]
    idxb = [(idx_d0, idx_s0), (idx_d1, idx_s1)]

    def do_chunk(j, p):
        idxd, idxs = idxb[p]
        for g in range(C // 16):
            idxd[pl.ds(16 * g, 16)] = sdx[pl.ds(j * C + 16 * g, 16)]
            idxs[pl.ds(16 * g, 16)] = ssx[pl.ds(j * C + 16 * g, 16)]

        pltpu.async_copy(td_hbm.at[idxd], ra, sem_g)
        pltpu.async_copy(ts_hbm.at[idxs], rb, sem_g)
        pltpu.make_async_copy(zero_hbm.at[pl.ds(0, C)], obuf, sem_sm).wait()
        pltpu.make_async_copy(td_hbm.at[pl.ds(0, C)], ra, sem_g).wait()
        pltpu.make_async_copy(td_hbm.at[pl.ds(0, C)], rb, sem_g).wait()

        def edge_grp(i, c2):
            for u in range(UNROLL):
                e = UNROLL * i + u
                ecast = jnp.full((16,), e, jnp.int32)
                dstv = plsc.load_gather(idxd, [ecast])
                parts = []
                for j2 in range(8):
                    t = ra[e, pl.ds(16 * j2, 16)] + rb[e, pl.ds(16 * j2, 16)]
                    t = jnp.where(t > 0.0, t, 0.01 * t)
                    parts.append(t * wa2v[j2])
                acc = ((parts[0] + parts[1]) + (parts[2] + parts[3])) + \
                      ((parts[4] + parts[5]) + (parts[6] + parts[7]))
                for pm in perms:
                    acc = acc + acc.at[pm].get(mode="promise_in_bounds")
                exv = jnp.exp(acc)
                plsc.addupdate_scatter(den_v, [dstv], exv, mask=lane0)
                for j2 in range(8):
                    u2 = (ra[e, pl.ds(D + 16 * j2, 16)]
                          + rb[e, pl.ds(D + 16 * j2, 16)])
                    obuf[e, pl.ds(16 * j2, 16)] = jnp.maximum(u2, 0.0) * exv
            return c2

        lax.fori_loop(0, C // UNROLL, edge_grp, 0)

        pltpu.async_copy(obuf, msg_tab.at[idxd], sem_sm, add=True)

    pltpu.sync_copy(dst_hbm.at[pl.ds(wid * EPW, C)], idx_d1)
    pltpu.async_copy(obuf, msg_tab.at[idx_d1], sem_sm, add=True)

    def super_body(s, carry):
        base = wid * EPW + s * SCSZ
        pltpu.sync_copy(dst_hbm.at[pl.ds(base, SCSZ)], sdx)
        pltpu.sync_copy(src_hbm.at[pl.ds(base, SCSZ)], ssx)

        def pair_body(i, c2):
            do_chunk(2 * i, 0)
            do_chunk(2 * i + 1, 1)
            return c2

        lax.fori_loop(0, SUPER // 2, pair_body, 0)
        return carry

    lax.fori_loop(0, NCHUNK // SUPER, super_body, 0)
    pltpu.make_async_copy(zero_hbm.at[pl.ds(0, C)], obuf, sem_sm).wait()

    plsc.subcore_barrier()
    pltpu.sync_copy(msg_tab.at[pl.ds(sid * (NPAD // NS), NPAD // NS)],
                    out_msg.at[cid, pl.ds(sid * (NPAD // NS), NPAD // NS)])
    pltpu.sync_copy(den_v, out_den.at[cid, sid])


def _dsum_body(den, out):
    out[...] = jnp.sum(den[...], axis=0)


_dsum = pl.pallas_call(
    _dsum_body,
    grid=(1,),
    in_specs=[pl.BlockSpec((NW, NPAD // D, D), lambda i: (0, 0, 0))],
    out_specs=pl.BlockSpec((NPAD // D, D), lambda i: (0, 0)),
    out_shape=jax.ShapeDtypeStruct((NPAD // D, D), jnp.float32),
)


def _update_body(msg, den, h, wih, whh, bih, bhh, wout, bout, out):
    a = msg[0] + msg[1]
    d = den[...]
    agg = a / (d + 1e-9)
    gi = agg @ wih[...] + bih[...]
    gh = h[...] @ whh[...] + bhh[...]
    r = jax.nn.sigmoid(gi[:, :D] + gh[:, :D])
    z = jax.nn.sigmoid(gi[:, D:2 * D] + gh[:, D:2 * D])
    n = jnp.tanh(gi[:, 2 * D:] + r * gh[:, 2 * D:])
    hn = (1.0 - z) * n + z * h[...]
    out[...] = jnp.maximum(hn @ wout[...] + bout[...], 0.0)


_update = pl.pallas_call(
    _update_body,
    grid=(N // BN,),
    in_specs=[
        pl.BlockSpec((NC, BN, D), lambda i: (0, i, 0)),
        pl.BlockSpec((BN, 1), lambda i: (i, 0)),
        pl.BlockSpec((BN, D), lambda i: (i, 0)),
        pl.BlockSpec((D, 3 * D), lambda i: (0, 0)),
        pl.BlockSpec((D, 3 * D), lambda i: (0, 0)),
        pl.BlockSpec((1, 3 * D), lambda i: (0, 0)),
        pl.BlockSpec((1, 3 * D), lambda i: (0, 0)),
        pl.BlockSpec((D, D), lambda i: (0, 0)),
        pl.BlockSpec((1, D), lambda i: (0, 0)),
    ],
    out_specs=pl.BlockSpec((BN, D), lambda i: (i, 0)),
    out_shape=jax.ShapeDtypeStruct((N, D), jnp.float32),
)


def kernel(obj_loc, obj_ske, obj_type, h, edge_index, W_e1, b_e1, W_e2, b_e2,
           Wa1, Wa2, Ww, W_ih, W_hh, b_ih, b_hh, W_out, b_out):
    ei = edge_index.astype(jnp.int32)
    pad = jnp.full((EP - E,), N + 100, jnp.int32)
    src_i = jnp.concatenate([ei[0], pad])
    dst_i = jnp.concatenate([ei[1], pad])

    w1 = W_e1.T
    w1s, w1t, w1l = w1[:D], w1[D:D + 16], w1[D + 16:]
    b1 = b_e1.reshape(1, D)
    b2 = b_e2.reshape(1, D)
    mqd = (Wa1[:, :D] + Wa1[:, D:]).T
    mqs = -Wa1[:, D:].T
    mpd = (Ww[:, :D] + Ww[:, D:]).T
    mps = -Ww[:, D:].T

    td, ts = _proj(obj_ske, obj_type, obj_loc, h, w1s, w1t, w1l,
                   b1, W_e2.T, b2, mqd, mqs, mpd, mps)
    padrows = jnp.zeros((NPAD - N, 2 * D), jnp.float32)
    td = jnp.concatenate([td, padrows])
    ts = jnp.concatenate([ts, padrows])

    wa2 = Wa2.reshape(D)
    zeros = jnp.zeros((NPAD, D), jnp.float32)
    msg, den = _edge_pass(td, ts, dst_i, src_i, wa2, zeros)
    dsum = _dsum(den.reshape(NW, NPAD // D, D))
    den2 = dsum.reshape(NPAD, 1)

    return _update(msg, den2, h, W_ih.T, W_hh.T, b_ih.reshape(1, 3 * D),
                   b_hh.reshape(1, 3 * D), W_out.T, b_out.reshape(1, D))
